# baseline (speedup 1.0000x reference)
import jax
import jax.numpy as jnp
import numpy as np
from jax.experimental import pallas as pl

EMBED = 768
HEADS = 12
DH = 64
LQ = 1024
LK = 4096
BATCH = 4
SCALE = 1.0 / np.sqrt(DH)
HPP = 6


def _attn_kernel(x_ref, wq_ref, wk_ref, wv_ref, wo_ref, bq_ref, cv_ref, o_ref):
    hp = pl.program_id(1)
    xb = x_ref[0]
    qp = jnp.dot(xb[:LQ], wq_ref[0], preferred_element_type=jnp.float32)
    qp = qp + bq_ref[pl.ds(hp, 1), :]
    kp = jnp.dot(xb, wk_ref[0], preferred_element_type=jnp.float32)
    vp = jnp.dot(xb, wv_ref[0], preferred_element_type=jnp.float32)
    ohs = []
    for i in range(HPP):
        q = qp[:, i * DH:(i + 1) * DH]
        k = kp[:, i * DH:(i + 1) * DH]
        v = vp[:, i * DH:(i + 1) * DH]
        s = jnp.dot(q, k.T, preferred_element_type=jnp.float32)
        p = jnp.exp(s)
        denom = jnp.sum(p, axis=-1, keepdims=True)
        pv = jnp.dot(p, v, preferred_element_type=jnp.float32)
        ohs.append(pv / denom)
    oh = jnp.concatenate(ohs, axis=1)
    contrib = jnp.dot(oh, wo_ref[0], preferred_element_type=jnp.float32)

    @pl.when(hp == 0)
    def _init():
        o_ref[...] = cv_ref[...] + contrib[None]

    @pl.when(hp != 0)
    def _acc():
        o_ref[...] += contrib[None]


def kernel(x, sorted_index, Wq, bq, Wk, bk, Wv, bv, Wo, bo):
    del bk
    cv = (bv @ Wo + bo).reshape(1, EMBED)
    npair = HEADS // HPP
    dp = HPP * DH
    bq2 = (bq * SCALE).reshape(npair, dp)
    xc = x.astype(jnp.bfloat16)
    wq3 = (Wq * SCALE).reshape(EMBED, npair, dp).transpose(1, 0, 2).astype(jnp.bfloat16)
    wk3 = Wk.reshape(EMBED, npair, dp).transpose(1, 0, 2).astype(jnp.bfloat16)
    wv3 = Wv.reshape(EMBED, npair, dp).transpose(1, 0, 2).astype(jnp.bfloat16)
    wo3 = Wo.reshape(npair, dp, EMBED)
    out = pl.pallas_call(
        _attn_kernel,
        grid=(BATCH, npair),
        in_specs=[
            pl.BlockSpec((1, LK, EMBED), lambda b, h: (b, 0, 0)),
            pl.BlockSpec((1, EMBED, dp), lambda b, h: (h, 0, 0)),
            pl.BlockSpec((1, EMBED, dp), lambda b, h: (h, 0, 0)),
            pl.BlockSpec((1, EMBED, dp), lambda b, h: (h, 0, 0)),
            pl.BlockSpec((1, dp, EMBED), lambda b, h: (h, 0, 0)),
            pl.BlockSpec((npair, dp), lambda b, h: (0, 0)),
            pl.BlockSpec((1, EMBED), lambda b, h: (0, 0)),
        ],
        out_specs=pl.BlockSpec((1, LQ, EMBED), lambda b, h: (b, 0, 0)),
        out_shape=jax.ShapeDtypeStruct((BATCH, LQ, EMBED), jnp.float32),
    )(xc, wq3, wk3, wv3, wo3, bq2, cv)
    return (out, sorted_index)


# speedup vs baseline: 1.2912x; 1.2912x over previous
import jax
import jax.numpy as jnp
import numpy as np
from jax.experimental import pallas as pl

EMBED = 768
HEADS = 12
DH = 64
LQ = 1024
LK = 4096
BATCH = 4
SCALE = 1.0 / np.sqrt(DH)
HPP = 4


def _attn_kernel(x_ref, wq_ref, wk_ref, wv_ref, wo_ref, bq_ref, cv_ref, o_ref):
    hp = pl.program_id(1)
    xb = x_ref[0]
    qp = jnp.dot(xb[:LQ], wq_ref[0], preferred_element_type=jnp.float32)
    qp = qp + bq_ref[pl.ds(hp, 1), :]
    kp = jnp.dot(xb, wk_ref[0], preferred_element_type=jnp.float32)
    vp = jnp.dot(xb, wv_ref[0], preferred_element_type=jnp.float32)
    ohs = []
    for i in range(HPP):
        q = qp[:, i * DH:(i + 1) * DH]
        k = kp[:, i * DH:(i + 1) * DH]
        v = vp[:, i * DH:(i + 1) * DH]
        s = jnp.dot(q, k.T, preferred_element_type=jnp.float32)
        p = jnp.exp(s)
        denom = jnp.sum(p, axis=-1, keepdims=True)
        pv = jnp.dot(p, v, preferred_element_type=jnp.float32)
        ohs.append(pv / denom)
    oh = jnp.concatenate(ohs, axis=1)
    contrib = jnp.dot(oh, wo_ref[0], preferred_element_type=jnp.float32)

    @pl.when(hp == 0)
    def _init():
        o_ref[...] = cv_ref[...] + contrib[None]

    @pl.when(hp != 0)
    def _acc():
        o_ref[...] += contrib[None]


def kernel(x, sorted_index, Wq, bq, Wk, bk, Wv, bv, Wo, bo):
    del bk
    cv = (bv @ Wo + bo).reshape(1, EMBED)
    npair = HEADS // HPP
    dp = HPP * DH
    bq2 = (bq * SCALE).reshape(npair, dp)
    xc = x.astype(jnp.bfloat16)
    wq3 = (Wq * SCALE).reshape(EMBED, npair, dp).transpose(1, 0, 2).astype(jnp.bfloat16)
    wk3 = Wk.reshape(EMBED, npair, dp).transpose(1, 0, 2).astype(jnp.bfloat16)
    wv3 = Wv.reshape(EMBED, npair, dp).transpose(1, 0, 2).astype(jnp.bfloat16)
    wo3 = Wo.reshape(npair, dp, EMBED)
    out = pl.pallas_call(
        _attn_kernel,
        grid=(BATCH, npair),
        in_specs=[
            pl.BlockSpec((1, LK, EMBED), lambda b, h: (b, 0, 0)),
            pl.BlockSpec((1, EMBED, dp), lambda b, h: (h, 0, 0)),
            pl.BlockSpec((1, EMBED, dp), lambda b, h: (h, 0, 0)),
            pl.BlockSpec((1, EMBED, dp), lambda b, h: (h, 0, 0)),
            pl.BlockSpec((1, dp, EMBED), lambda b, h: (h, 0, 0)),
            pl.BlockSpec((npair, dp), lambda b, h: (0, 0)),
            pl.BlockSpec((1, EMBED), lambda b, h: (0, 0)),
        ],
        out_specs=pl.BlockSpec((1, LQ, EMBED), lambda b, h: (b, 0, 0)),
        out_shape=jax.ShapeDtypeStruct((BATCH, LQ, EMBED), jnp.float32),
    )(xc, wq3, wk3, wv3, wo3, bq2, cv)
    return (out, sorted_index)


# speedup vs baseline: 1.3634x; 1.0559x over previous
import jax
import jax.numpy as jnp
import numpy as np
from jax.experimental import pallas as pl

EMBED = 768
HEADS = 12
DH = 64
LQ = 1024
LK = 4096
BATCH = 4
SCALE = 1.0 / np.sqrt(DH)
HPP = 4


def _attn_kernel(x_ref, wq_ref, wk_ref, wv_ref, wo_ref, bq_ref, cv_ref, o_ref):
    hp = pl.program_id(1)
    xb = x_ref[0]
    qp = jnp.dot(xb[:LQ], wq_ref[0], preferred_element_type=jnp.float32)
    qp = qp + bq_ref[pl.ds(hp, 1), :]
    kp = jnp.dot(xb, wk_ref[0], preferred_element_type=jnp.float32)
    vp = jnp.dot(xb, wv_ref[0], preferred_element_type=jnp.float32)
    ohs = []
    for i in range(HPP):
        q = qp[:, i * DH:(i + 1) * DH]
        k = kp[:, i * DH:(i + 1) * DH]
        v = vp[:, i * DH:(i + 1) * DH]
        s = jnp.dot(q, k.T, preferred_element_type=jnp.float32)
        p = jnp.exp(s)
        denom = jnp.sum(p, axis=-1, keepdims=True)
        pv = jnp.dot(p, v, preferred_element_type=jnp.float32)
        ohs.append(pv / denom)
    oh = jnp.concatenate(ohs, axis=1)
    contrib = jnp.dot(oh, wo_ref[0], preferred_element_type=jnp.float32)

    @pl.when(hp == 0)
    def _init():
        o_ref[...] = cv_ref[...] + contrib[None]

    @pl.when(hp != 0)
    def _acc():
        o_ref[...] += contrib[None]


def kernel(x, sorted_index, Wq, bq, Wk, bk, Wv, bv, Wo, bo):
    del bk
    cv = (bv @ Wo + bo).reshape(1, EMBED)
    npair = HEADS // HPP
    dp = HPP * DH
    bq2 = (bq * SCALE).reshape(npair, dp)
    wq3 = (Wq * SCALE).reshape(EMBED, npair, dp).transpose(1, 0, 2)
    wk3 = Wk.reshape(EMBED, npair, dp).transpose(1, 0, 2)
    wv3 = Wv.reshape(EMBED, npair, dp).transpose(1, 0, 2)
    wo3 = Wo.reshape(npair, dp, EMBED)
    out = pl.pallas_call(
        _attn_kernel,
        grid=(BATCH, npair),
        in_specs=[
            pl.BlockSpec((1, LK, EMBED), lambda b, h: (b, 0, 0)),
            pl.BlockSpec((1, EMBED, dp), lambda b, h: (h, 0, 0)),
            pl.BlockSpec((1, EMBED, dp), lambda b, h: (h, 0, 0)),
            pl.BlockSpec((1, EMBED, dp), lambda b, h: (h, 0, 0)),
            pl.BlockSpec((1, dp, EMBED), lambda b, h: (h, 0, 0)),
            pl.BlockSpec((npair, dp), lambda b, h: (0, 0)),
            pl.BlockSpec((1, EMBED), lambda b, h: (0, 0)),
        ],
        out_specs=pl.BlockSpec((1, LQ, EMBED), lambda b, h: (b, 0, 0)),
        out_shape=jax.ShapeDtypeStruct((BATCH, LQ, EMBED), jnp.float32),
    )(x, wq3, wk3, wv3, wo3, bq2, cv)
    return (out, sorted_index)


# speedup vs baseline: 1.3673x; 1.0028x over previous
import jax
import jax.numpy as jnp
import numpy as np
from jax.experimental import pallas as pl
from jax.experimental.pallas import tpu as pltpu

EMBED = 768
HEADS = 12
DH = 64
LQ = 1024
LK = 4096
BATCH = 4
SCALE = 1.0 / np.sqrt(DH)
HPP = 4


def _attn_kernel(x_ref, wq_ref, wk_ref, wv_ref, wo_ref, bq_ref, cv_ref, o_ref):
    hp = pl.program_id(1)
    xb = x_ref[0]
    qp = jnp.dot(xb[:LQ], wq_ref[0], preferred_element_type=jnp.float32)
    qp = qp + bq_ref[pl.ds(hp, 1), :]
    kp = jnp.dot(xb, wk_ref[0], preferred_element_type=jnp.float32)
    vp = jnp.dot(xb, wv_ref[0], preferred_element_type=jnp.float32)
    ohs = []
    for i in range(HPP):
        q = qp[:, i * DH:(i + 1) * DH]
        k = kp[:, i * DH:(i + 1) * DH]
        v = vp[:, i * DH:(i + 1) * DH]
        s = jnp.dot(q, k.T, preferred_element_type=jnp.float32)
        p = jnp.exp(s)
        denom = jnp.sum(p, axis=-1, keepdims=True)
        pv = jnp.dot(p, v, preferred_element_type=jnp.float32)
        ohs.append(pv / denom)
    oh = jnp.concatenate(ohs, axis=1)
    contrib = jnp.dot(oh, wo_ref[0], preferred_element_type=jnp.float32)

    @pl.when(hp == 0)
    def _init():
        o_ref[...] = cv_ref[...] + contrib[None]

    @pl.when(hp != 0)
    def _acc():
        o_ref[...] += contrib[None]


def kernel(x, sorted_index, Wq, bq, Wk, bk, Wv, bv, Wo, bo):
    del bk
    cv = (bv @ Wo + bo).reshape(1, EMBED)
    npair = HEADS // HPP
    dp = HPP * DH
    bq2 = (bq * SCALE).reshape(npair, dp)
    wq3 = (Wq * SCALE).reshape(EMBED, npair, dp).transpose(1, 0, 2)
    wk3 = Wk.reshape(EMBED, npair, dp).transpose(1, 0, 2)
    wv3 = Wv.reshape(EMBED, npair, dp).transpose(1, 0, 2)
    wo3 = Wo.reshape(npair, dp, EMBED)
    out = pl.pallas_call(
        _attn_kernel,
        grid=(BATCH, npair),
        in_specs=[
            pl.BlockSpec((1, LK, EMBED), lambda b, h: (b, 0, 0)),
            pl.BlockSpec((1, EMBED, dp), lambda b, h: (h, 0, 0)),
            pl.BlockSpec((1, EMBED, dp), lambda b, h: (h, 0, 0)),
            pl.BlockSpec((1, EMBED, dp), lambda b, h: (h, 0, 0)),
            pl.BlockSpec((1, dp, EMBED), lambda b, h: (h, 0, 0)),
            pl.BlockSpec((npair, dp), lambda b, h: (0, 0)),
            pl.BlockSpec((1, EMBED), lambda b, h: (0, 0)),
        ],
        out_specs=pl.BlockSpec((1, LQ, EMBED), lambda b, h: (b, 0, 0)),
        out_shape=jax.ShapeDtypeStruct((BATCH, LQ, EMBED), jnp.float32),
        compiler_params=pltpu.CompilerParams(
            dimension_semantics=("parallel", "arbitrary")),
    )(x, wq3, wk3, wv3, wo3, bq2, cv)
    return (out, sorted_index)


# speedup vs baseline: 1.4428x; 1.0553x over previous
import jax
import jax.numpy as jnp
import numpy as np
from jax.experimental import pallas as pl
from jax.experimental.pallas import tpu as pltpu

EMBED = 768
HEADS = 12
DH = 64
LQ = 1024
LK = 4096
BATCH = 4
SCALE = 1.0 / np.sqrt(DH)
HPP = 4
NCH = 2


def _attn_kernel(x_ref, wq_ref, wk_ref, wv_ref, wo_ref, bq_ref, cv_ref, o_ref):
    hp = pl.program_id(1)
    xb = x_ref[0]
    qp = jnp.dot(xb[:LQ], wq_ref[0], preferred_element_type=jnp.float32)
    qp = qp + bq_ref[pl.ds(hp, 1), :]
    kp = jnp.dot(xb, wk_ref[0], preferred_element_type=jnp.float32)
    vp = jnp.dot(xb, wv_ref[0], preferred_element_type=jnp.float32)
    ohs = []
    for i in range(HPP):
        q = qp[:, i * DH:(i + 1) * DH]
        k = kp[:, i * DH:(i + 1) * DH]
        v = vp[:, i * DH:(i + 1) * DH]
        parts = []
        for c in range(NCH):
            qc = q[c * (LQ // NCH):(c + 1) * (LQ // NCH), :]
            s = jnp.dot(qc, k.T, preferred_element_type=jnp.float32)
            p = jnp.exp(s)
            denom = jnp.sum(p, axis=-1, keepdims=True)
            pv = jnp.dot(p, v, preferred_element_type=jnp.float32)
            parts.append(pv / denom)
        ohs.append(jnp.concatenate(parts, axis=0) if NCH > 1 else parts[0])
    oh = jnp.concatenate(ohs, axis=1)
    contrib = jnp.dot(oh, wo_ref[0], preferred_element_type=jnp.float32)

    @pl.when(hp == 0)
    def _init():
        o_ref[...] = cv_ref[...] + contrib[None]

    @pl.when(hp != 0)
    def _acc():
        o_ref[...] += contrib[None]


def kernel(x, sorted_index, Wq, bq, Wk, bk, Wv, bv, Wo, bo):
    del bk
    cv = (bv @ Wo + bo).reshape(1, EMBED)
    npair = HEADS // HPP
    dp = HPP * DH
    bq2 = (bq * SCALE).reshape(npair, dp)
    wq3 = (Wq * SCALE).reshape(EMBED, npair, dp).transpose(1, 0, 2)
    wk3 = Wk.reshape(EMBED, npair, dp).transpose(1, 0, 2)
    wv3 = Wv.reshape(EMBED, npair, dp).transpose(1, 0, 2)
    wo3 = Wo.reshape(npair, dp, EMBED)
    out = pl.pallas_call(
        _attn_kernel,
        grid=(BATCH, npair),
        in_specs=[
            pl.BlockSpec((1, LK, EMBED), lambda b, h: (b, 0, 0)),
            pl.BlockSpec((1, EMBED, dp), lambda b, h: (h, 0, 0)),
            pl.BlockSpec((1, EMBED, dp), lambda b, h: (h, 0, 0)),
            pl.BlockSpec((1, EMBED, dp), lambda b, h: (h, 0, 0)),
            pl.BlockSpec((1, dp, EMBED), lambda b, h: (h, 0, 0)),
            pl.BlockSpec((npair, dp), lambda b, h: (0, 0)),
            pl.BlockSpec((1, EMBED), lambda b, h: (0, 0)),
        ],
        out_specs=pl.BlockSpec((1, LQ, EMBED), lambda b, h: (b, 0, 0)),
        out_shape=jax.ShapeDtypeStruct((BATCH, LQ, EMBED), jnp.float32),
        compiler_params=pltpu.CompilerParams(
            dimension_semantics=("parallel", "arbitrary")),
    )(x, wq3, wk3, wv3, wo3, bq2, cv)
    return (out, sorted_index)
